# baseline (speedup 1.0000x reference)
import functools

import jax
import jax.numpy as jnp
from jax import lax
from jax.experimental import pallas as pl
from jax.experimental.pallas import tpu as pltpu
from jax.experimental.pallas import tpu_sc as plsc

DIM = 64
NS = 16384
NT = 26
NW = 32
S_PER_W = NS // NW
SB = 128
NSB = S_PER_W // SB
L = 16
RB = 4
TP = 136
TOTAL = NT * NSB


def _sc_gather_t(table, idx_t):
    mesh = plsc.VectorSubcoreMesh(core_axis_name="c", subcore_axis_name="s")

    @functools.partial(
        pl.kernel,
        out_type=jax.ShapeDtypeStruct((NT, 8, 128, 8, 128), jnp.float32),
        mesh=mesh,
        compiler_params=pltpu.CompilerParams(
            use_tc_tiling_on_sc=False,
            needs_layout_passes=False,
            disable_bounds_checks=True,
        ),
        scratch_types=[
            pltpu.VMEM((NT, S_PER_W), jnp.int32),
            pltpu.VMEM((RB, SB, DIM), jnp.float32),
            pltpu.VMEM((2, 8, 8, TP), jnp.float32),
            pltpu.SemaphoreType.DMA((RB,)),
            pltpu.SemaphoreType.DMA((2,)),
        ],
    )
    def k(table_hbm, idx_hbm, out_hbm, idx_v, rows_v, t_v, gsem, osem):
        wid = lax.axis_index("s") * 2 + lax.axis_index("c")
        s0 = wid * S_PER_W
        sh0 = wid * NSB

        for t in range(NT):
            pltpu.sync_copy(
                idx_hbm.at[pl.ds(t * NS + s0, S_PER_W)], idx_v.at[t]
            )

        def issue_gather(i, rb):
            t = i // NSB
            sb = lax.rem(i, NSB)
            pltpu.async_copy(
                table_hbm.at[idx_v.at[t, pl.ds(sb * SB, SB)]],
                rows_v.at[rb],
                gsem.at[rb],
            )

        def wait_gather(rb):
            pltpu.make_async_copy(
                table_hbm.at[idx_v.at[0, pl.ds(0, SB)]],
                rows_v.at[rb],
                gsem.at[rb],
            ).wait()

        def issue_store(t, sh, tb):
            pltpu.async_copy(
                t_v.at[tb, :, :, pl.ds(0, 128)],
                out_hbm.at[t, :, sh],
                osem.at[tb],
            )

        def wait_store(tb):
            pltpu.make_async_copy(
                t_v.at[tb, :, :, pl.ds(0, 128)],
                out_hbm.at[0, :, 0],
                osem.at[tb],
            ).wait()

        iota = lax.iota(jnp.int32, L)
        zero16 = iota * 0
        flat_c = [(16 * j + iota) * TP for j in range(DIM // L)]

        def transpose_block(rb, tb):
            rows = rows_v.at[rb]
            tbuf = t_v.at[tb]

            def load_s(s):
                return [rows[s, pl.ds(j * L, L)] for j in range(DIM // L)]

            def store_s(s, vs):
                svec = jnp.full((L,), 0, jnp.int32) + s
                for j, v in enumerate(vs):
                    plsc.store_scatter(tbuf, [zero16, zero16, flat_c[j] + svec], v)

            def s_body(s, carry):
                nxt = load_s(s + 1)
                store_s(s, carry)
                return nxt

            vs0 = load_s(0)
            last = lax.fori_loop(0, SB - 1, s_body, vs0, unroll=8)
            store_s(SB - 1, last)

        for p in range(RB - 1):
            issue_gather(jnp.int32(p), p)

        def tbody(t, _):
            base = t * NSB
            for sb in range(NSB):
                i = base + sb
                rb = sb
                tb = sb % 2
                wait_gather(rb)
                if sb >= 2:
                    wait_store(tb)
                else:
                    @pl.when(t > 0)
                    def _():
                        wait_store(tb)
                transpose_block(rb, tb)

                @pl.when(i < TOTAL - (RB - 1))
                def _():
                    issue_gather(i + (RB - 1), (sb + RB - 1) % RB)

                issue_store(t, sh0 + sb, tb)
            return 0

        lax.fori_loop(0, NT, tbody, 0)
        wait_store(0)
        wait_store(1)

    return k(table, idx_t)


def kernel(regime_idx, table):
    idx_flat = regime_idx.T.astype(jnp.int32).reshape(-1)
    a5 = _sc_gather_t(table, idx_flat)
    return jnp.transpose(a5, (2, 4, 0, 1, 3)).reshape(NS, NT, DIM)


# speedup vs baseline: 1.0628x; 1.0628x over previous
import functools

import jax
import jax.numpy as jnp
from jax import lax
from jax.experimental import pallas as pl
from jax.experimental.pallas import tpu as pltpu
from jax.experimental.pallas import tpu_sc as plsc

DIM = 64
NS = 16384
NT = 26
NW = 32
S_PER_W = NS // NW
SB = 128
NSB = S_PER_W // SB
L = 16
RB = 4
TP = 136
TOTAL = NT * NSB


def _sc_gather_t(table, idx_t):
    mesh = plsc.VectorSubcoreMesh(core_axis_name="c", subcore_axis_name="s")

    @functools.partial(
        pl.kernel,
        out_type=jax.ShapeDtypeStruct((NT, 8, 128, 8, 128), jnp.float32),
        mesh=mesh,
        compiler_params=pltpu.CompilerParams(
            use_tc_tiling_on_sc=False,
            needs_layout_passes=False,
            disable_bounds_checks=True,
        ),
        scratch_types=[
            pltpu.VMEM((NT, S_PER_W), jnp.int32),
            pltpu.VMEM((RB, SB, DIM), jnp.float32),
            pltpu.VMEM((2, 8, 8, TP), jnp.float32),
            pltpu.SemaphoreType.DMA((RB,)),
            pltpu.SemaphoreType.DMA((2,)),
        ],
    )
    def k(table_hbm, idx_hbm, out_hbm, idx_v, rows_v, t_v, gsem, osem):
        wid = lax.axis_index("s") * 2 + lax.axis_index("c")
        s0 = wid * S_PER_W
        sh0 = wid * NSB

        pltpu.sync_copy(idx_hbm.at[:, pl.ds(s0, S_PER_W)], idx_v)

        def issue_gather(i, rb):
            t = i // NSB
            sb = lax.rem(i, NSB)
            pltpu.async_copy(
                table_hbm.at[idx_v.at[t, pl.ds(sb * SB, SB)]],
                rows_v.at[rb],
                gsem.at[rb],
            )

        def wait_gather(rb):
            pltpu.make_async_copy(
                table_hbm.at[idx_v.at[0, pl.ds(0, SB)]],
                rows_v.at[rb],
                gsem.at[rb],
            ).wait()

        def issue_store(t, sh, tb):
            pltpu.async_copy(
                t_v.at[tb, :, :, pl.ds(0, 128)],
                out_hbm.at[t, :, sh],
                osem.at[tb],
            )

        def wait_store(tb):
            pltpu.make_async_copy(
                t_v.at[tb, :, :, pl.ds(0, 128)],
                out_hbm.at[0, :, 0],
                osem.at[tb],
            ).wait()

        iota = lax.iota(jnp.int32, L)
        zero16 = iota * 0
        flat_c = [(16 * j + iota) * TP for j in range(DIM // L)]

        def transpose_block(rb, tb):
            rows = rows_v.at[rb]
            tbuf = t_v.at[tb]

            def load_s(s):
                return [rows[s, pl.ds(j * L, L)] for j in range(DIM // L)]

            def store_s(s, vs):
                svec = jnp.full((L,), 0, jnp.int32) + s
                for j, v in enumerate(vs):
                    plsc.store_scatter(tbuf, [zero16, zero16, flat_c[j] + svec], v)

            def s_body(s, carry):
                nxt = load_s(s + 1)
                store_s(s, carry)
                return nxt

            vs0 = load_s(0)
            last = lax.fori_loop(0, SB - 1, s_body, vs0, unroll=8)
            store_s(SB - 1, last)

        for p in range(RB - 1):
            issue_gather(jnp.int32(p), p)

        def tbody(t, _):
            base = t * NSB
            for sb in range(NSB):
                i = base + sb
                rb = sb
                tb = sb % 2
                wait_gather(rb)
                if sb >= 2:
                    wait_store(tb)
                else:
                    @pl.when(t > 0)
                    def _():
                        wait_store(tb)
                transpose_block(rb, tb)

                @pl.when(i < TOTAL - (RB - 1))
                def _():
                    issue_gather(i + (RB - 1), (sb + RB - 1) % RB)

                issue_store(t, sh0 + sb, tb)
            return 0

        lax.fori_loop(0, NT, tbody, 0)
        wait_store(0)
        wait_store(1)

    return k(table, idx_t)


def kernel(regime_idx, table):
    idx_t = regime_idx.T.astype(jnp.int32)
    a5 = _sc_gather_t(table, idx_t)
    return jnp.transpose(a5, (2, 4, 0, 1, 3)).reshape(NS, NT, DIM)
